# baseline (speedup 1.0000x reference)
import functools

import jax
import jax.numpy as jnp
from jax import lax
from jax.experimental import pallas as pl
from jax.experimental.pallas import tpu as pltpu
from jax.experimental.pallas import tpu_sc as plsc

N = 50000
NP = 50048
E = 800000
EB = 128
NBLK = E // EB
NS = 16
NC = 2
RPT = NP // NS
ZROWS = 136

@functools.cache
def _mesh():
    return plsc.VectorSubcoreMesh(core_axis_name="c", subcore_axis_name="s",
                                  num_cores=NC, num_subcores=NS)


def _zero_acc(acc, zbuf, s, width):
    for r in range(ZROWS):
        for c16 in range(width // 16):
            zbuf[r, pl.ds(c16 * 16, 16)] = jnp.zeros((16,), jnp.float32)
    base = s * RPT
    for j in range(RPT // ZROWS):
        pltpu.sync_copy(zbuf, acc.at[pl.ds(base + j * ZROWS, ZROWS)])


def _edge_pipeline(ei, tables, acc, idxbuf, rowsbuf, isems, gsems, ssems,
                   base, stride, nticks8, cnt=None, ones_v=None, csems=None):
    def _gather(pred, ref, bb, rb):
        if pred is None:
            pltpu.async_copy(ref.at[idxbuf.at[bb, 0]], rowsbuf.at[rb],
                             gsems[rb])
        else:
            @pl.when(pred)
            def _():
                pltpu.async_copy(ref.at[idxbuf.at[bb, 0]], rowsbuf.at[rb],
                                 gsems[rb])

    def tick(t, b):
        tw = t - 6
        bw = (b - 6) % 8
        rw = (b - 6) % 4
        blk_w = base + stride * tw
        @pl.when((tw >= 0) & (blk_w < NBLK))
        def _():
            pltpu.make_async_copy(rowsbuf.at[rw],
                                  acc.at[idxbuf.at[bw, 1]], ssems[rw]).wait()
            if cnt is not None:
                pltpu.make_async_copy(ones_v, cnt.at[idxbuf.at[bw, 1]],
                                      csems[rw]).wait()

        blk_a = base + stride * t
        @pl.when(blk_a < NBLK)
        def _():
            pltpu.async_copy(ei.at[blk_a], idxbuf.at[b], isems[b])

        tb = t - 2
        bb = (b - 2) % 8
        blk_b = base + stride * tb
        @pl.when((tb >= 0) & (blk_b < NBLK))
        def _():
            pltpu.make_async_copy(ei.at[0], idxbuf.at[bb], isems[bb]).wait()
            for pred, ref in tables:
                _gather(pred, ref, bb, (b - 2) % 4)

        tcx = t - 4
        bc = (b - 4) % 8
        rc = (b - 4) % 4
        blk_c = base + stride * tcx
        @pl.when((tcx >= 0) & (blk_c < NBLK))
        def _():
            pltpu.make_async_copy(tables[0][1].at[pl.ds(0, EB)],
                                  rowsbuf.at[rc], gsems[rc]).wait()
            pltpu.async_copy(rowsbuf.at[rc], acc.at[idxbuf.at[bc, 1]],
                             ssems[rc], add=True)
            if cnt is not None:
                pltpu.async_copy(ones_v, cnt.at[idxbuf.at[bc, 1]],
                                 csems[rc], add=True)

    def body(g, carry):
        for b in range(8):
            tick(g * 8 + b, b)
        return carry

    lax.fori_loop(0, nticks8, body, 0)


def _zero_acc32(acc, rowsbuf, s):
    for r in range(EB):
        for c16 in range(2):
            rowsbuf[0, r, pl.ds(c16 * 16, 16)] = jnp.zeros((16,), jnp.float32)
    base = s * RPT
    for j in range(RPT // EB):
        pltpu.sync_copy(rowsbuf.at[0], acc.at[pl.ds(base + j * EB, EB)])
    rem = RPT % EB
    if rem:
        pltpu.sync_copy(rowsbuf.at[0, pl.ds(0, rem)],
                        acc.at[pl.ds(base + (RPT // EB) * EB, rem)])


def _sem_scratch(n=20):
    return [pltpu.SemaphoreType.DMA] * n


def _split_sems(sems):
    return list(sems[:8]), list(sems[8:16]), list(sems[16:20])


def _sc_agg1(ei, x16_hbm, part0, part1, cnt0, cnt1,
             idxbuf, rowsbuf, ones_v, zbuf, zbuf1, acc, cnt, *sems):
    c = lax.axis_index("c")
    s = lax.axis_index("s")
    w = c * NS + s
    isems, gsems, ssems = _split_sems(sems)
    csems = list(sems[20:24])

    _zero_acc(acc, zbuf, s, 16)
    for c16 in range(EB // 16):
        ones_v[pl.ds(c16 * 16, 16)] = jnp.ones((16,), jnp.float32)
    for z16 in range(3136 // 16):
        zbuf1[pl.ds(z16 * 16, 16)] = jnp.zeros((16,), jnp.float32)
    pltpu.sync_copy(zbuf1.at[pl.ds(0, RPT)], cnt.at[pl.ds(s * RPT, RPT)])
    plsc.subcore_barrier()

    _edge_pipeline(ei, [(None, x16_hbm)], acc, idxbuf, rowsbuf, isems, gsems,
                   ssems, base=w, stride=NC * NS, nticks8=26, cnt=cnt,
                   ones_v=ones_v, csems=csems)
    plsc.subcore_barrier()

    sl = pl.ds(s * RPT, RPT)
    @pl.when(c == 0)
    def _():
        pltpu.sync_copy(acc.at[sl], part0.at[sl])
        pltpu.sync_copy(cnt.at[sl], cnt0.at[sl])
    @pl.when(c == 1)
    def _():
        pltpu.sync_copy(acc.at[sl], part1.at[sl])
        pltpu.sync_copy(cnt.at[sl], cnt1.at[sl])


def _run_agg1(ei3, x16):
    f = pl.kernel(
        _sc_agg1,
        out_type=[
            jax.ShapeDtypeStruct((NP, 16), jnp.float32),
            jax.ShapeDtypeStruct((NP, 16), jnp.float32),
            jax.ShapeDtypeStruct((NP,), jnp.float32),
            jax.ShapeDtypeStruct((NP,), jnp.float32),
        ],
        mesh=_mesh(),
        compiler_params=pltpu.CompilerParams(use_tc_tiling_on_sc=False),
        scratch_types=[
            pltpu.VMEM((8, 2, EB), jnp.int32),
            pltpu.VMEM((4, EB, 16), jnp.float32),
            pltpu.VMEM((EB,), jnp.float32),
            pltpu.VMEM((ZROWS, 16), jnp.float32),
            pltpu.VMEM((3136,), jnp.float32),
            pltpu.VMEM_SHARED((NP, 16), jnp.float32),
            pltpu.VMEM_SHARED((NP,), jnp.float32),
        ] + _sem_scratch(24),
    )
    return f(ei3, x16)


def _sc_agg2(ei, h1c0, h1c1, out0, out1,
             idxbuf, rowsbuf, acc, *sems):
    c = lax.axis_index("c")
    s = lax.axis_index("s")
    isems, gsems, ssems = _split_sems(sems)

    _zero_acc32(acc, rowsbuf, s)
    plsc.subcore_barrier()

    _edge_pipeline(ei, [(c == 0, h1c0), (c == 1, h1c1)], acc, idxbuf,
                   rowsbuf, isems, gsems, ssems, base=s, stride=NS,
                   nticks8=50)
    plsc.subcore_barrier()

    sl = pl.ds(s * RPT, RPT)
    @pl.when(c == 0)
    def _():
        pltpu.sync_copy(acc.at[sl], out0.at[sl])
    @pl.when(c == 1)
    def _():
        pltpu.sync_copy(acc.at[sl], out1.at[sl])


def _run_agg2(ei3, h1c0, h1c1):
    f = pl.kernel(
        _sc_agg2,
        out_type=[
            jax.ShapeDtypeStruct((NP, 32), jnp.float32),
            jax.ShapeDtypeStruct((NP, 32), jnp.float32),
        ],
        mesh=_mesh(),
        compiler_params=pltpu.CompilerParams(use_tc_tiling_on_sc=False),
        scratch_types=[
            pltpu.VMEM((8, 2, EB), jnp.int32),
            pltpu.VMEM((4, EB, 32), jnp.float32),
            pltpu.VMEM_SHARED((NP, 32), jnp.float32),
        ] + _sem_scratch(),
    )
    return f(ei3, h1c0, h1c1)


def _sc_agg3(ei, h2c0, h2c1, h2c2, h2c3,
             out0, out1, out2, out3,
             idxbuf, rowsbuf, acc, *sems):
    c = lax.axis_index("c")
    s = lax.axis_index("s")
    isems, gsems, ssems = _split_sems(sems)
    sl = pl.ds(s * RPT, RPT)
    srcs = ((h2c0, h2c2), (h2c1, h2c3))
    outs = ((out0, out2), (out1, out3))

    for k in range(2):
        _zero_acc32(acc, rowsbuf, s)
        plsc.subcore_barrier()

        _edge_pipeline(ei, [(c == 0, srcs[k][0]), (c == 1, srcs[k][1])],
                       acc, idxbuf, rowsbuf, isems, gsems, ssems,
                       base=s, stride=NS, nticks8=50)
        plsc.subcore_barrier()

        @pl.when(c == 0)
        def _():
            pltpu.sync_copy(acc.at[sl], outs[k][0].at[sl])
        @pl.when(c == 1)
        def _():
            pltpu.sync_copy(acc.at[sl], outs[k][1].at[sl])
        plsc.subcore_barrier()


def _run_agg3(ei3, h2c):
    f = pl.kernel(
        _sc_agg3,
        out_type=[jax.ShapeDtypeStruct((NP, 32), jnp.float32)] * 4,
        mesh=_mesh(),
        compiler_params=pltpu.CompilerParams(use_tc_tiling_on_sc=False),
        scratch_types=[
            pltpu.VMEM((8, 2, EB), jnp.int32),
            pltpu.VMEM((4, EB, 32), jnp.float32),
            pltpu.VMEM_SHARED((NP, 32), jnp.float32),
        ] + _sem_scratch(),
    )
    return f(ei3, *h2c)


BN = 2048
_GRID = (NP + BN - 1) // BN


def _unpack(p, w):
    g = 128 // w
    x = p.shape[0]
    return jnp.stack([p[:, w * k:w * (k + 1)] for k in range(g)],
                     axis=1).reshape(x * g, w)


def _pack(a, w):
    g = 128 // w
    m = a.shape[0]
    v = a.reshape(m // g, g, w)
    return jnp.concatenate([v[:, k, :] for k in range(g)], axis=1)


def _bspec(*shape):
    nd = len(shape)
    return pl.BlockSpec(shape, lambda i, _nd=nd: (i,) + (0,) * (_nd - 1))


def _wspec(*shape):
    nd = len(shape)
    return pl.BlockSpec(shape, lambda i, _nd=nd: (0,) * _nd)


def _tc_layer1(p0, p1, c0, c1, x16, w_l, b_l, w_r, h1c0, h1c1, inv_ref):
    cnt = c0[...] + c1[...]
    inv = 1.0 / jnp.maximum(cnt, 1.0)
    inv_ref[...] = inv
    agg = _unpack(p0[...] + p1[...], 16)
    x = _unpack(x16[...], 16)
    out = (jnp.dot(agg, w_l[...], preferred_element_type=jnp.float32)
           * inv[:, None]
           + jnp.dot(x, w_r[...], preferred_element_type=jnp.float32)
           + b_l[...])
    nrm = jnp.sqrt(jnp.sum(out * out, axis=-1, keepdims=True))
    out = out / jnp.maximum(nrm, 1e-12)
    out = jnp.maximum(out, 0.0)
    h1c0[...] = _pack(out[:, :32], 32)
    h1c1[...] = _pack(out[:, 32:], 32)


def _run_layer1(p0, p1, c0, c1, x16, w_l, b_l, w_r):
    return pl.pallas_call(
        _tc_layer1,
        grid=(_GRID,),
        in_specs=[
            _bspec(BN * 16 // 128, 128), _bspec(BN * 16 // 128, 128),
            _bspec(BN), _bspec(BN),
            _bspec(BN * 16 // 128, 128),
            _wspec(16, 64), _wspec(1, 64), _wspec(16, 64),
        ],
        out_specs=[_bspec(BN * 32 // 128, 128), _bspec(BN * 32 // 128, 128),
                   _bspec(BN)],
        out_shape=[
            jax.ShapeDtypeStruct((NP * 32 // 128, 128), jnp.float32),
            jax.ShapeDtypeStruct((NP * 32 // 128, 128), jnp.float32),
            jax.ShapeDtypeStruct((NP,), jnp.float32),
        ],
    )(p0, p1, c0, c1, x16, w_l, b_l, w_r)


def _tc_layer2(a0, a1, inv, h1c0, h1c1, w_l, b_l, w_r, o0, o1, o2, o3):
    agg = jnp.concatenate([_unpack(a0[...], 32), _unpack(a1[...], 32)], axis=1)
    h1 = jnp.concatenate([_unpack(h1c0[...], 32), _unpack(h1c1[...], 32)],
                         axis=1)
    out = (jnp.dot(agg, w_l[...], preferred_element_type=jnp.float32)
           * inv[...][:, None]
           + jnp.dot(h1, w_r[...], preferred_element_type=jnp.float32)
           + b_l[...])
    out = jnp.maximum(out, 0.0)
    o0[...] = _pack(out[:, :32], 32)
    o1[...] = _pack(out[:, 32:64], 32)
    o2[...] = _pack(out[:, 64:96], 32)
    o3[...] = _pack(out[:, 96:], 32)


def _run_layer2(a0, a1, inv, h1c0, h1c1, w_l, b_l, w_r):
    pk = _bspec(BN * 32 // 128, 128)
    return pl.pallas_call(
        _tc_layer2,
        grid=(_GRID,),
        in_specs=[
            pk, pk, _bspec(BN), pk, pk,
            _wspec(64, 128), _wspec(1, 128), _wspec(64, 128),
        ],
        out_specs=[pk] * 4,
        out_shape=[jax.ShapeDtypeStruct((NP * 32 // 128, 128),
                                        jnp.float32)] * 4,
    )(a0, a1, inv, h1c0, h1c1, w_l, b_l, w_r)


def _tc_layer3(a0, a1, a2, a3, inv, h0, h1, h2, h3,
               w_l, b_l, w_r, l1w, l1b, l2w, l2b, l3w, l3b, out):
    agg = jnp.concatenate([_unpack(a0[...], 32), _unpack(a1[...], 32),
                           _unpack(a2[...], 32), _unpack(a3[...], 32)],
                          axis=1)
    h = jnp.concatenate([_unpack(h0[...], 32), _unpack(h1[...], 32),
                         _unpack(h2[...], 32), _unpack(h3[...], 32)], axis=1)
    z = (jnp.dot(agg, w_l[...], preferred_element_type=jnp.float32)
         * inv[...][:, None]
         + jnp.dot(h, w_r[...], preferred_element_type=jnp.float32)
         + b_l[...])
    z = jnp.maximum(
        jnp.dot(z, l1w[...], preferred_element_type=jnp.float32) + l1b[...], 0.0)
    z = jnp.maximum(
        jnp.dot(z, l2w[...], preferred_element_type=jnp.float32) + l2b[...], 0.0)
    lg = jnp.dot(z, l3w[...], preferred_element_type=jnp.float32) + l3b[...]
    m = jnp.max(lg, axis=-1, keepdims=True)
    lse = m + jnp.log(jnp.sum(jnp.exp(lg - m), axis=-1, keepdims=True))
    out[...] = lg - lse


def _run_layer3(a, inv, h2c, w_l, b_l, w_r, l1w, l1b, l2w, l2b, l3w, l3b):
    pk = _bspec(BN * 32 // 128, 128)
    return pl.pallas_call(
        _tc_layer3,
        grid=(_GRID,),
        in_specs=[
            pk, pk, pk, pk,
            _bspec(BN),
            pk, pk, pk, pk,
            _wspec(128, 128), _wspec(1, 128), _wspec(128, 128),
            _wspec(128, 128), _wspec(1, 128),
            _wspec(128, 64), _wspec(1, 64),
            _wspec(64, 8), _wspec(1, 8),
        ],
        out_specs=[_bspec(BN, 8)],
        out_shape=[jax.ShapeDtypeStruct((NP, 8), jnp.float32)],
    )(*a, inv, *h2c, w_l, b_l, w_r, l1w, l1b, l2w, l2b, l3w, l3b)[0]


def kernel(x, edge_index, W1l, b1l, W1r, W2l, b2l, W2r, W3l, b3l, W3r,
           L1W, L1b, L2W, L2b, L3W, L3b):
    ei3 = edge_index.reshape(2, NBLK, EB).transpose(1, 0, 2)
    x16 = jnp.pad(x, ((0, NP - x.shape[0]), (0, 16 - x.shape[1])))

    w1l = jnp.pad(W1l.T, ((0, 16 - W1l.shape[1]), (0, 0)))
    w1r = jnp.pad(W1r.T, ((0, 16 - W1r.shape[1]), (0, 0)))
    l3w = jnp.pad(L3W.T, ((0, 0), (0, 8 - L3W.shape[0])))
    l3b = jnp.pad(L3b, (0, 8 - L3b.shape[0]),
                  constant_values=-1e30).reshape(1, 8)

    def pk16(v):
        return v.reshape(NP * 16 // 128, 128)

    def pk32(v):
        return v.reshape(NP * 32 // 128, 128)

    p0, p1, c0, c1 = _run_agg1(ei3, x16)
    h1c0, h1c1, inv = _run_layer1(pk16(p0), pk16(p1), c0, c1, pk16(x16),
                                  w1l, b1l.reshape(1, 64), w1r)
    a20, a21 = _run_agg2(ei3, h1c0.reshape(NP, 32), h1c1.reshape(NP, 32))
    h2c = _run_layer2(pk32(a20), pk32(a21), inv, h1c0, h1c1, W2l.T,
                      b2l.reshape(1, 128), W2r.T)
    a3 = _run_agg3(ei3, [h.reshape(NP, 32) for h in h2c])
    out8 = _run_layer3([pk32(a) for a in a3], inv, h2c, W3l.T,
                       b3l.reshape(1, 128), W3r.T, L1W.T,
                       L1b.reshape(1, 128), L2W.T, L2b.reshape(1, 64),
                       l3w, l3b)
    return out8[:N, :3]


# speedup vs baseline: 1.4205x; 1.4205x over previous
import functools

import jax
import jax.numpy as jnp
from jax import lax
from jax.experimental import pallas as pl
from jax.experimental.pallas import tpu as pltpu
from jax.experimental.pallas import tpu_sc as plsc

N = 50000
NP = 50048
E = 800000
EB = 200
NBLK = E // EB
NS = 16
NC = 2
RPT = NP // NS
ZROWS = 136

@functools.cache
def _mesh():
    return plsc.VectorSubcoreMesh(core_axis_name="c", subcore_axis_name="s",
                                  num_cores=NC, num_subcores=NS)


def _zero_acc(acc, zbuf, s, width):
    for r in range(ZROWS):
        for c16 in range(width // 16):
            zbuf[r, pl.ds(c16 * 16, 16)] = jnp.zeros((16,), jnp.float32)
    base = s * RPT
    for j in range(RPT // ZROWS):
        pltpu.sync_copy(zbuf, acc.at[pl.ds(base + j * ZROWS, ZROWS)])


def _edge_pipeline(ei, tables, acc, idxbuf, rowsbuf, isems, gsems, ssems,
                   base, stride, nticks8, cnt=None, ones_v=None, csems=None):
    def _gather(pred, ref, bb, rb):
        if pred is None:
            pltpu.async_copy(ref.at[idxbuf.at[bb, 0]],
                             rowsbuf.at[rb], gsems[rb])
        else:
            @pl.when(pred)
            def _():
                pltpu.async_copy(ref.at[idxbuf.at[bb, 0]],
                                 rowsbuf.at[rb], gsems[rb])

    def tick(t, b):
        tw = t - 6
        bw = (b - 6) % 8
        rw = (b - 6) % 4
        blk_w = base + stride * tw
        @pl.when((tw >= 0) & (blk_w < NBLK))
        def _():
            pltpu.make_async_copy(rowsbuf.at[rw],
                                  acc.at[idxbuf.at[bw, 1]],
                                  ssems[rw]).wait()
            if cnt is not None:
                pltpu.make_async_copy(ones_v, cnt.at[idxbuf.at[bw, 1]],
                                      csems[rw]).wait()

        blk_a = base + stride * t
        @pl.when(blk_a < NBLK)
        def _():
            pltpu.async_copy(ei.at[blk_a], idxbuf.at[b], isems[b])

        tb = t - 2
        bb = (b - 2) % 8
        blk_b = base + stride * tb
        @pl.when((tb >= 0) & (blk_b < NBLK))
        def _():
            pltpu.make_async_copy(ei.at[0], idxbuf.at[bb], isems[bb]).wait()
            for pred, ref in tables:
                _gather(pred, ref, bb, (b - 2) % 4)

        tcx = t - 4
        bc = (b - 4) % 8
        rc = (b - 4) % 4
        blk_c = base + stride * tcx
        @pl.when((tcx >= 0) & (blk_c < NBLK))
        def _():
            pltpu.make_async_copy(tables[0][1].at[pl.ds(0, EB)],
                                  rowsbuf.at[rc], gsems[rc]).wait()
            pltpu.async_copy(rowsbuf.at[rc],
                             acc.at[idxbuf.at[bc, 1]],
                             ssems[rc], add=True)
            if cnt is not None:
                pltpu.async_copy(ones_v, cnt.at[idxbuf.at[bc, 1]],
                                 csems[rc], add=True)

    def body(g, carry):
        for b in range(8):
            tick(g * 8 + b, b)
        return carry

    lax.fori_loop(0, nticks8, body, 0)


def _zero_acc32(acc, rowsbuf, s):
    for r in range(EB):
        for c16 in range(2):
            rowsbuf[0, r, pl.ds(c16 * 16, 16)] = jnp.zeros(
                (16,), jnp.float32)
    base = s * RPT
    for j in range(RPT // EB):
        pltpu.sync_copy(rowsbuf.at[0], acc.at[pl.ds(base + j * EB, EB)])
    rem = RPT % EB
    if rem:
        pltpu.sync_copy(rowsbuf.at[0, pl.ds(0, rem)],
                        acc.at[pl.ds(base + (RPT // EB) * EB, rem)])


def _sem_scratch(n=20):
    return [pltpu.SemaphoreType.DMA] * n


def _split_sems(sems):
    return list(sems[:8]), list(sems[8:16]), list(sems[16:20])


def _sc_agg1(ei, x16_hbm, part0, part1, cnt0, cnt1,
             idxbuf, rowsbuf, ones_v, zbuf, zbuf1, acc, cnt, *sems):
    c = lax.axis_index("c")
    s = lax.axis_index("s")
    w = c * NS + s
    isems, gsems, ssems = _split_sems(sems)
    csems = list(sems[20:24])

    _zero_acc(acc, zbuf, s, 16)
    for oj in range(EB // 4):
        ones_v[pl.ds(oj * 4, 4)] = jnp.ones((4,), jnp.float32)
    for z16 in range(3136 // 16):
        zbuf1[pl.ds(z16 * 16, 16)] = jnp.zeros((16,), jnp.float32)
    pltpu.sync_copy(zbuf1.at[pl.ds(0, RPT)], cnt.at[pl.ds(s * RPT, RPT)])
    plsc.subcore_barrier()

    _edge_pipeline(ei, [(None, x16_hbm)], acc, idxbuf, rowsbuf, isems, gsems,
                   ssems, base=w, stride=NC * NS, nticks8=17, cnt=cnt,
                   ones_v=ones_v, csems=csems)
    plsc.subcore_barrier()

    sl = pl.ds(s * RPT, RPT)
    @pl.when(c == 0)
    def _():
        pltpu.sync_copy(acc.at[sl], part0.at[sl])
        pltpu.sync_copy(cnt.at[sl], cnt0.at[sl])
    @pl.when(c == 1)
    def _():
        pltpu.sync_copy(acc.at[sl], part1.at[sl])
        pltpu.sync_copy(cnt.at[sl], cnt1.at[sl])


def _run_agg1(ei3, x16):
    f = pl.kernel(
        _sc_agg1,
        out_type=[
            jax.ShapeDtypeStruct((NP, 16), jnp.float32),
            jax.ShapeDtypeStruct((NP, 16), jnp.float32),
            jax.ShapeDtypeStruct((NP,), jnp.float32),
            jax.ShapeDtypeStruct((NP,), jnp.float32),
        ],
        mesh=_mesh(),
        compiler_params=pltpu.CompilerParams(use_tc_tiling_on_sc=False),
        scratch_types=[
            pltpu.VMEM((8, 2, EB), jnp.int32),
            pltpu.VMEM((4, EB, 16), jnp.float32),
            pltpu.VMEM((EB,), jnp.float32),
            pltpu.VMEM((ZROWS, 16), jnp.float32),
            pltpu.VMEM((3136,), jnp.float32),
            pltpu.VMEM_SHARED((NP, 16), jnp.float32),
            pltpu.VMEM_SHARED((NP,), jnp.float32),
        ] + _sem_scratch(24),
    )
    return f(ei3, x16)


def _sc_agg2(ei, h1c0, h1c1, out0, out1,
             idxbuf, rowsbuf, acc, *sems):
    c = lax.axis_index("c")
    s = lax.axis_index("s")
    isems, gsems, ssems = _split_sems(sems)

    _zero_acc32(acc, rowsbuf, s)
    plsc.subcore_barrier()

    _edge_pipeline(ei, [(c == 0, h1c0), (c == 1, h1c1)], acc, idxbuf,
                   rowsbuf, isems, gsems, ssems, base=s, stride=NS,
                   nticks8=32)
    plsc.subcore_barrier()

    sl = pl.ds(s * RPT, RPT)
    @pl.when(c == 0)
    def _():
        pltpu.sync_copy(acc.at[sl], out0.at[sl])
    @pl.when(c == 1)
    def _():
        pltpu.sync_copy(acc.at[sl], out1.at[sl])


def _run_agg2(ei3, h1c0, h1c1):
    f = pl.kernel(
        _sc_agg2,
        out_type=[
            jax.ShapeDtypeStruct((NP, 32), jnp.float32),
            jax.ShapeDtypeStruct((NP, 32), jnp.float32),
        ],
        mesh=_mesh(),
        compiler_params=pltpu.CompilerParams(use_tc_tiling_on_sc=False),
        scratch_types=[
            pltpu.VMEM((8, 2, EB), jnp.int32),
            pltpu.VMEM((4, EB, 32), jnp.float32),
            pltpu.VMEM_SHARED((NP, 32), jnp.float32),
        ] + _sem_scratch(),
    )
    return f(ei3, h1c0, h1c1)


def _sc_agg3(ei, h2c0, h2c1, h2c2, h2c3,
             out0, out1, out2, out3,
             idxbuf, rowsbuf, acc, *sems):
    c = lax.axis_index("c")
    s = lax.axis_index("s")
    isems, gsems, ssems = _split_sems(sems)
    sl = pl.ds(s * RPT, RPT)
    srcs = ((h2c0, h2c2), (h2c1, h2c3))
    outs = ((out0, out2), (out1, out3))

    for k in range(2):
        _zero_acc32(acc, rowsbuf, s)
        plsc.subcore_barrier()

        _edge_pipeline(ei, [(c == 0, srcs[k][0]), (c == 1, srcs[k][1])],
                       acc, idxbuf, rowsbuf, isems, gsems, ssems,
                       base=s, stride=NS, nticks8=32)
        plsc.subcore_barrier()

        @pl.when(c == 0)
        def _():
            pltpu.sync_copy(acc.at[sl], outs[k][0].at[sl])
        @pl.when(c == 1)
        def _():
            pltpu.sync_copy(acc.at[sl], outs[k][1].at[sl])
        plsc.subcore_barrier()


def _run_agg3(ei3, h2c):
    f = pl.kernel(
        _sc_agg3,
        out_type=[jax.ShapeDtypeStruct((NP, 32), jnp.float32)] * 4,
        mesh=_mesh(),
        compiler_params=pltpu.CompilerParams(use_tc_tiling_on_sc=False),
        scratch_types=[
            pltpu.VMEM((8, 2, EB), jnp.int32),
            pltpu.VMEM((4, EB, 32), jnp.float32),
            pltpu.VMEM_SHARED((NP, 32), jnp.float32),
        ] + _sem_scratch(),
    )
    return f(ei3, *h2c)


BN = 2048
_GRID = (NP + BN - 1) // BN


def _bspec(*shape):
    nd = len(shape)
    return pl.BlockSpec(shape, lambda i, _nd=nd: (i,) + (0,) * (_nd - 1))


def _wspec(*shape):
    nd = len(shape)
    return pl.BlockSpec(shape, lambda i, _nd=nd: (0,) * _nd)


def _tc_layer1(p0, p1, c0, c1, x16, w_l, b_l, w_r, h1c0, h1c1, inv_ref):
    cnt = c0[...] + c1[...]
    inv = 1.0 / jnp.maximum(cnt, 1.0)
    inv_ref[...] = inv
    mean = (p0[...] + p1[...]) * inv[:, None]
    out = (jnp.dot(mean, w_l[...], preferred_element_type=jnp.float32)
           + jnp.dot(x16[...], w_r[...], preferred_element_type=jnp.float32)
           + b_l[...])
    nrm = jnp.sqrt(jnp.sum(out * out, axis=-1, keepdims=True))
    out = out / jnp.maximum(nrm, 1e-12)
    out = jnp.maximum(out, 0.0)
    h1c0[...] = out[:, :32]
    h1c1[...] = out[:, 32:]


def _run_layer1(p0, p1, c0, c1, x16, w_l, b_l, w_r):
    return pl.pallas_call(
        _tc_layer1,
        grid=(_GRID,),
        in_specs=[
            _bspec(BN, 16), _bspec(BN, 16), _bspec(BN), _bspec(BN),
            _bspec(BN, 16), _wspec(16, 64), _wspec(1, 64), _wspec(16, 64),
        ],
        out_specs=[_bspec(BN, 32), _bspec(BN, 32), _bspec(BN)],
        out_shape=[
            jax.ShapeDtypeStruct((NP, 32), jnp.float32),
            jax.ShapeDtypeStruct((NP, 32), jnp.float32),
            jax.ShapeDtypeStruct((NP,), jnp.float32),
        ],
    )(p0, p1, c0, c1, x16, w_l, b_l, w_r)


def _tc_layer2(a0, a1, inv, h1c0, h1c1, w_l, b_l, w_r, o0, o1, o2, o3):
    agg = jnp.concatenate([a0[...], a1[...]], axis=1)
    mean = agg * inv[...][:, None]
    h1 = jnp.concatenate([h1c0[...], h1c1[...]], axis=1)
    out = (jnp.dot(mean, w_l[...], preferred_element_type=jnp.float32)
           + jnp.dot(h1, w_r[...], preferred_element_type=jnp.float32)
           + b_l[...])
    out = jnp.maximum(out, 0.0)
    o0[...] = out[:, :32]
    o1[...] = out[:, 32:64]
    o2[...] = out[:, 64:96]
    o3[...] = out[:, 96:]


def _run_layer2(a0, a1, inv, h1c0, h1c1, w_l, b_l, w_r):
    return pl.pallas_call(
        _tc_layer2,
        grid=(_GRID,),
        in_specs=[
            _bspec(BN, 32), _bspec(BN, 32), _bspec(BN),
            _bspec(BN, 32), _bspec(BN, 32),
            _wspec(64, 128), _wspec(1, 128), _wspec(64, 128),
        ],
        out_specs=[_bspec(BN, 32)] * 4,
        out_shape=[jax.ShapeDtypeStruct((NP, 32), jnp.float32)] * 4,
    )(a0, a1, inv, h1c0, h1c1, w_l, b_l, w_r)


def _tc_layer3(a0, a1, a2, a3, inv, h0, h1, h2, h3,
               w_l, b_l, w_r, l1w, l1b, l2w, l2b, l3w, l3b, out):
    agg = jnp.concatenate([a0[...], a1[...], a2[...], a3[...]], axis=1)
    mean = agg * inv[...][:, None]
    h = jnp.concatenate([h0[...], h1[...], h2[...], h3[...]], axis=1)
    z = (jnp.dot(mean, w_l[...], preferred_element_type=jnp.float32)
         + jnp.dot(h, w_r[...], preferred_element_type=jnp.float32)
         + b_l[...])
    z = jnp.maximum(
        jnp.dot(z, l1w[...], preferred_element_type=jnp.float32) + l1b[...], 0.0)
    z = jnp.maximum(
        jnp.dot(z, l2w[...], preferred_element_type=jnp.float32) + l2b[...], 0.0)
    lg = jnp.dot(z, l3w[...], preferred_element_type=jnp.float32) + l3b[...]
    m = jnp.max(lg, axis=-1, keepdims=True)
    lse = m + jnp.log(jnp.sum(jnp.exp(lg - m), axis=-1, keepdims=True))
    out[...] = lg - lse


def _run_layer3(a, inv, h2c, w_l, b_l, w_r, l1w, l1b, l2w, l2b, l3w, l3b):
    return pl.pallas_call(
        _tc_layer3,
        grid=(_GRID,),
        in_specs=[
            _bspec(BN, 32), _bspec(BN, 32), _bspec(BN, 32), _bspec(BN, 32),
            _bspec(BN),
            _bspec(BN, 32), _bspec(BN, 32), _bspec(BN, 32), _bspec(BN, 32),
            _wspec(128, 128), _wspec(1, 128), _wspec(128, 128),
            _wspec(128, 128), _wspec(1, 128),
            _wspec(128, 64), _wspec(1, 64),
            _wspec(64, 8), _wspec(1, 8),
        ],
        out_specs=[_bspec(BN, 8)],
        out_shape=[jax.ShapeDtypeStruct((NP, 8), jnp.float32)],
    )(*a, inv, *h2c, w_l, b_l, w_r, l1w, l1b, l2w, l2b, l3w, l3b)[0]


def kernel(x, edge_index, W1l, b1l, W1r, W2l, b2l, W2r, W3l, b3l, W3r,
           L1W, L1b, L2W, L2b, L3W, L3b):
    ei3 = edge_index.reshape(2, NBLK, EB).transpose(1, 0, 2)
    x16 = jnp.pad(x, ((0, NP - x.shape[0]), (0, 16 - x.shape[1])))

    w1l = jnp.pad(W1l.T, ((0, 16 - W1l.shape[1]), (0, 0)))
    w1r = jnp.pad(W1r.T, ((0, 16 - W1r.shape[1]), (0, 0)))
    l3w = jnp.pad(L3W.T, ((0, 0), (0, 8 - L3W.shape[0])))
    l3b = jnp.pad(L3b, (0, 8 - L3b.shape[0]),
                  constant_values=-1e30).reshape(1, 8)

    p0, p1, c0, c1 = _run_agg1(ei3, x16)
    h1c0, h1c1, inv = _run_layer1(p0, p1, c0, c1, x16, w1l,
                                  b1l.reshape(1, 64), w1r)
    a20, a21 = _run_agg2(ei3, h1c0, h1c1)
    h2c = _run_layer2(a20, a21, inv, h1c0, h1c1, W2l.T,
                      b2l.reshape(1, 128), W2r.T)
    a3 = _run_agg3(ei3, h2c)
    out8 = _run_layer3(a3, inv, h2c, W3l.T, b3l.reshape(1, 128), W3r.T,
                       L1W.T, L1b.reshape(1, 128), L2W.T, L2b.reshape(1, 64),
                       l3w, l3b)
    return out8[:N, :3]
